# baseline (speedup 1.0000x reference)
import functools

import jax
import jax.numpy as jnp
from jax import lax
from jax.experimental import pallas as pl
from jax.experimental.pallas import tpu as pltpu
from jax.experimental.pallas import tpu_sc as plsc

_B, _T = 4096, 50
_DIMS = (64, 64, 32, 32)
_WPAD = (65, 65, 33, 33)
_OFFS = (0, 64, 128, 160)
_DSUM = 192
_OTW = 129
_NC, _NS = 2, 16
_NW = _NC * _NS
_BLK = _B // _NW
_L = 16

_mesh = plsc.VectorSubcoreMesh(core_axis_name="c", subcore_axis_name="s")


@functools.partial(
    pl.kernel,
    out_type=jax.ShapeDtypeStruct((_T, _DSUM, _B), jnp.float32),
    mesh=_mesh,
    compiler_params=pltpu.CompilerParams(
        use_tc_tiling_on_sc=False, needs_layout_passes=False),
    scratch_types=[
        pltpu.VMEM((_T, _BLK), jnp.int32),
        pltpu.VMEM((_T, _BLK), jnp.int32),
        pltpu.VMEM((_T, _BLK), jnp.int32),
        pltpu.VMEM((_T, _BLK), jnp.int32),
        pltpu.VMEM((2, _BLK, _WPAD[0]), jnp.float32),
        pltpu.VMEM((2, _BLK, _WPAD[1]), jnp.float32),
        pltpu.VMEM((2, _BLK, _WPAD[2]), jnp.float32),
        pltpu.VMEM((2, _BLK, _WPAD[3]), jnp.float32),
        pltpu.VMEM((_DSUM, _OTW), jnp.float32),
        pltpu.SemaphoreType.DMA,
        pltpu.SemaphoreType.DMA,
        pltpu.SemaphoreType.DMA,
    ],
)
def _emb_gather(x0_h, x1_h, x2_h, x3_h, w0_h, w1_h, w2_h, w3_h, out_h,
                i0, i1, i2, i3, s0, s1, s2, s3, ot, gsem0, gsem1, osem):
    wid = lax.axis_index("s") * _NC + lax.axis_index("c")
    b0 = wid * _BLK

    pltpu.sync_copy(x0_h.at[:, pl.ds(b0, _BLK)], i0)
    pltpu.sync_copy(x1_h.at[:, pl.ds(b0, _BLK)], i1)
    pltpu.sync_copy(x2_h.at[:, pl.ds(b0, _BLK)], i2)
    pltpu.sync_copy(x3_h.at[:, pl.ds(b0, _BLK)], i3)

    idx_refs = (i0, i1, i2, i3)
    w_refs = (w0_h, w1_h, w2_h, w3_h)
    stages = (s0, s1, s2, s3)
    gsems = (gsem0, gsem1)

    def gather_copies(t, sl):
        for k in range(4):
            src = w_refs[k].at[idx_refs[k].at[t]]
            yield src, stages[k].at[sl], gsems[sl]

    def out_copy(t):
        yield ot.at[:, pl.ds(0, _BLK)], out_h.at[t, :, pl.ds(b0, _BLK)], osem

    def fire(copies):
        for src, dst, sem in copies:
            pltpu.async_copy(src, dst, sem)

    def drain(copies):
        for src, dst, sem in copies:
            pltpu.make_async_copy(src, dst, sem).wait()

    lane = lax.iota(jnp.int32, _L)
    rowidx = [[_OFFS[k] + q * _L + lane for q in range(_DIMS[k] // _L)]
              for k in range(4)]

    def transpose_into(sl):
        def b_body(b):
            cols = jnp.broadcast_to(b, (_L,)).astype(jnp.int32)
            for k in range(4):
                for q in range(_DIMS[k] // _L):
                    v = stages[k][sl, b, pl.ds(q * _L, _L)]
                    plsc.store_scatter(ot, [rowidx[k][q], cols], v)

        pl.loop(0, _BLK)(b_body)

    fire(gather_copies(0, 0))

    def pair_body(t):
        for sl in range(2):
            ct = t + sl

            @pl.when(ct + 1 <= _T - 1)
            def _():
                fire(gather_copies(ct + 1, 1 - sl))

            drain(gather_copies(ct, sl))

            @pl.when(ct >= 1)
            def _():
                drain(out_copy(ct - 1))

            transpose_into(sl)
            fire(out_copy(ct))

    pl.loop(0, _T, step=2)(pair_body)
    drain(out_copy(_T - 1))


def kernel(x0, x1, x2, x3, W0, W1, W2, W3):
    xs = [x.astype(jnp.int32).T for x in (x0, x1, x2, x3)]
    ws = [jnp.pad(w, ((0, 0), (0, wp - w.shape[1])))
          for w, wp in zip((W0, W1, W2, W3), _WPAD)]
    out_t = _emb_gather(xs[0], xs[1], xs[2], xs[3], ws[0], ws[1], ws[2], ws[3])
    return jnp.transpose(out_t, (2, 0, 1))


# speedup vs baseline: 1.2251x; 1.2251x over previous
import functools

import jax
import jax.numpy as jnp
from jax import lax
from jax.experimental import pallas as pl
from jax.experimental.pallas import tpu as pltpu
from jax.experimental.pallas import tpu_sc as plsc

_B, _T = 4096, 50
_DIMS = (64, 64, 32, 32)
_WPAD = (64, 64, 32, 32)
_OFFS = (0, 64, 128, 160)
_DSUM = 192
_OTW = 129
_NC, _NS = 2, 16
_NW = _NC * _NS
_BLK = _B // _NW
_L = 16

_mesh = plsc.VectorSubcoreMesh(core_axis_name="c", subcore_axis_name="s")


@functools.partial(
    pl.kernel,
    out_type=jax.ShapeDtypeStruct((_T, _DSUM, _B), jnp.float32),
    mesh=_mesh,
    compiler_params=pltpu.CompilerParams(
        use_tc_tiling_on_sc=False, needs_layout_passes=False),
    scratch_types=[
        pltpu.VMEM((_T, _BLK), jnp.int32),
        pltpu.VMEM((_T, _BLK), jnp.int32),
        pltpu.VMEM((_T, _BLK), jnp.int32),
        pltpu.VMEM((_T, _BLK), jnp.int32),
        pltpu.VMEM((2, _BLK, _WPAD[0]), jnp.float32),
        pltpu.VMEM((2, _BLK, _WPAD[1]), jnp.float32),
        pltpu.VMEM((2, _BLK, _WPAD[2]), jnp.float32),
        pltpu.VMEM((2, _BLK, _WPAD[3]), jnp.float32),
        pltpu.VMEM((_DSUM, _OTW), jnp.float32),
        pltpu.SemaphoreType.DMA,
        pltpu.SemaphoreType.DMA,
        pltpu.SemaphoreType.DMA,
    ],
)
def _emb_gather(x0_h, x1_h, x2_h, x3_h, w0_h, w1_h, w2_h, w3_h, out_h,
                i0, i1, i2, i3, s0, s1, s2, s3, ot, gsem0, gsem1, osem):
    wid = lax.axis_index("s") * _NC + lax.axis_index("c")
    b0 = wid * _BLK

    pltpu.sync_copy(x0_h.at[:, pl.ds(b0, _BLK)], i0)
    pltpu.sync_copy(x1_h.at[:, pl.ds(b0, _BLK)], i1)
    pltpu.sync_copy(x2_h.at[:, pl.ds(b0, _BLK)], i2)
    pltpu.sync_copy(x3_h.at[:, pl.ds(b0, _BLK)], i3)

    idx_refs = (i0, i1, i2, i3)
    w_refs = (w0_h, w1_h, w2_h, w3_h)
    stages = (s0, s1, s2, s3)
    gsems = (gsem0, gsem1)

    def gather_copies(t, sl):
        for k in range(4):
            src = w_refs[k].at[idx_refs[k].at[t]]
            yield src, stages[k].at[sl], gsems[sl]

    def out_copy(t):
        yield ot.at[:, pl.ds(0, _BLK)], out_h.at[t, :, pl.ds(b0, _BLK)], osem

    def fire(copies):
        for src, dst, sem in copies:
            pltpu.async_copy(src, dst, sem)

    def drain(copies):
        for src, dst, sem in copies:
            pltpu.make_async_copy(src, dst, sem).wait()

    lane = lax.iota(jnp.int32, _L)
    rowidx = [[_OFFS[k] + q * _L + lane for q in range(_DIMS[k] // _L)]
              for k in range(4)]

    def transpose_into(sl):
        def b_body(b):
            cols = jnp.broadcast_to(b, (_L,)).astype(jnp.int32)
            for k in range(4):
                for q in range(_DIMS[k] // _L):
                    v = stages[k][sl, b, pl.ds(q * _L, _L)]
                    plsc.store_scatter(ot, [rowidx[k][q], cols], v)

        pl.loop(0, _BLK)(b_body)

    fire(gather_copies(0, 0))

    def pair_body(t):
        for sl in range(2):
            ct = t + sl

            @pl.when(ct + 1 <= _T - 1)
            def _():
                fire(gather_copies(ct + 1, 1 - sl))

            drain(gather_copies(ct, sl))

            @pl.when(ct >= 1)
            def _():
                drain(out_copy(ct - 1))

            transpose_into(sl)
            fire(out_copy(ct))

    pl.loop(0, _T, step=2)(pair_body)
    drain(out_copy(_T - 1))


def kernel(x0, x1, x2, x3, W0, W1, W2, W3):
    xs = [x.astype(jnp.int32).T for x in (x0, x1, x2, x3)]
    out_t = _emb_gather(xs[0], xs[1], xs[2], xs[3], W0, W1, W2, W3)
    return jnp.transpose(out_t, (2, 0, 1))


# speedup vs baseline: 1.2374x; 1.0101x over previous
import functools

import jax
import jax.numpy as jnp
from jax import lax
from jax.experimental import pallas as pl
from jax.experimental.pallas import tpu as pltpu
from jax.experimental.pallas import tpu_sc as plsc

_B, _T = 4096, 50
_DIMS = (64, 64, 32, 32)
_WPAD = (64, 64, 32, 32)
_OFFS = (0, 64, 128, 160)
_DSUM = 192
_OTW = 129
_NC, _NS = 2, 16
_NW = _NC * _NS
_BLK = _B // _NW
_L = 16

_mesh = plsc.VectorSubcoreMesh(core_axis_name="c", subcore_axis_name="s")


@functools.partial(
    pl.kernel,
    out_type=jax.ShapeDtypeStruct((_T, _DSUM, _B), jnp.float32),
    mesh=_mesh,
    compiler_params=pltpu.CompilerParams(
        use_tc_tiling_on_sc=False, needs_layout_passes=False),
    scratch_types=[
        pltpu.VMEM((_T, _BLK), jnp.int32),
        pltpu.VMEM((_T, _BLK), jnp.int32),
        pltpu.VMEM((_T, _BLK), jnp.int32),
        pltpu.VMEM((_T, _BLK), jnp.int32),
        pltpu.VMEM((2, _BLK, _WPAD[0]), jnp.float32),
        pltpu.VMEM((2, _BLK, _WPAD[1]), jnp.float32),
        pltpu.VMEM((2, _BLK, _WPAD[2]), jnp.float32),
        pltpu.VMEM((2, _BLK, _WPAD[3]), jnp.float32),
        pltpu.VMEM((_DSUM, _OTW), jnp.float32),
        pltpu.SemaphoreType.DMA,
        pltpu.SemaphoreType.DMA,
        pltpu.SemaphoreType.DMA,
    ],
)
def _emb_gather(x0_h, x1_h, x2_h, x3_h, w0_h, w1_h, w2_h, w3_h, out_h,
                i0, i1, i2, i3, s0, s1, s2, s3, ot, gsem0, gsem1, osem):
    wid = lax.axis_index("s") * _NC + lax.axis_index("c")
    b0 = wid * _BLK

    pltpu.sync_copy(x0_h.at[:, pl.ds(b0, _BLK)], i0)
    pltpu.sync_copy(x1_h.at[:, pl.ds(b0, _BLK)], i1)
    pltpu.sync_copy(x2_h.at[:, pl.ds(b0, _BLK)], i2)
    pltpu.sync_copy(x3_h.at[:, pl.ds(b0, _BLK)], i3)

    idx_refs = (i0, i1, i2, i3)
    w_refs = (w0_h, w1_h, w2_h, w3_h)
    stages = (s0, s1, s2, s3)
    gsems = (gsem0, gsem1)

    def gather_copies(t, sl):
        for k in range(4):
            src = w_refs[k].at[idx_refs[k].at[t]]
            yield src, stages[k].at[sl], gsems[sl]

    def out_copy(t):
        yield ot.at[:, pl.ds(0, _BLK)], out_h.at[t, :, pl.ds(b0, _BLK)], osem

    def fire(copies):
        for src, dst, sem in copies:
            pltpu.async_copy(src, dst, sem)

    def drain(copies):
        for src, dst, sem in copies:
            pltpu.make_async_copy(src, dst, sem).wait()

    lane = lax.iota(jnp.int32, _L)
    zeros = jnp.zeros((_L,), jnp.int32)
    rowflat = [[(_OFFS[k] + q * _L + lane) * _OTW
                for q in range(_DIMS[k] // _L)] for k in range(4)]

    def transpose_into(sl):
        def b_body(b):
            for db in range(4):
                bb = b + db
                for k in range(4):
                    for q in range(_DIMS[k] // _L):
                        v = stages[k][sl, bb, pl.ds(q * _L, _L)]
                        plsc.store_scatter(ot, [zeros, rowflat[k][q] + bb], v)

        pl.loop(0, _BLK, step=4)(b_body)

    fire(gather_copies(0, 0))

    def pair_body(t):
        for sl in range(2):
            ct = t + sl

            @pl.when(ct + 1 <= _T - 1)
            def _():
                fire(gather_copies(ct + 1, 1 - sl))

            drain(gather_copies(ct, sl))

            @pl.when(ct >= 1)
            def _():
                drain(out_copy(ct - 1))

            transpose_into(sl)
            fire(out_copy(ct))

    pl.loop(0, _T, step=2)(pair_body)
    drain(out_copy(_T - 1))


def kernel(x0, x1, x2, x3, W0, W1, W2, W3):
    xs = [x.astype(jnp.int32).T for x in (x0, x1, x2, x3)]
    out_t = _emb_gather(xs[0], xs[1], xs[2], xs[3], W0, W1, W2, W3)
    return jnp.transpose(out_t, (2, 0, 1))
